# baseline (speedup 1.0000x reference)
import functools

import jax
import jax.numpy as jnp
from jax import lax
from jax.experimental import pallas as pl
from jax.experimental.pallas import tpu as pltpu
from jax.experimental.pallas import tpu_sc as plsc

_B = 16384
_V = 1000000
_D = 64
_NEG = 8
_NC = 2
_NS = 16
_NW = _NC * _NS
_CH = _B // _NW
_NCHUNK = 2 + _NEG
_NBUF = 3
_L = 16

_NFULL = _V // 128
_TAIL = _V - _NFULL * 128
_PERW = _NFULL // _NW
_EXTRA = _NFULL - _PERW * _NW
_MAXB = _PERW + 1


def _linearize_table(table_t):
    mesh = plsc.VectorSubcoreMesh(core_axis_name="c", subcore_axis_name="s")

    @functools.partial(
        pl.kernel,
        mesh=mesh,
        out_type=jax.ShapeDtypeStruct((_V * _D,), jnp.float32),
        scratch_types=(
            [pltpu.VMEM((_D, 128), jnp.float32) for _ in range(2)]
            + [pltpu.VMEM((128 * _D,), jnp.float32) for _ in range(2)]
            + [pltpu.VMEM((_D, _TAIL), jnp.float32)]
            + [pltpu.SemaphoreType.DMA for _ in range(4)]
        ),
        compiler_params=pltpu.CompilerParams(
            use_tc_tiling_on_sc=True, needs_layout_passes=False),
    )
    def body(tt_hbm, out_hbm, ib0, ib1, ob0, ob1, tb, is0, is1, os0, os1):
        ib, ob, isem, osem = (ib0, ib1), (ob0, ob1), (is0, is1), (os0, os1)
        wid = lax.axis_index("s") * _NC + lax.axis_index("c")
        extra = jnp.minimum(wid, _EXTRA)
        base = wid * _PERW + extra
        nblk = jnp.where(wid < _EXTRA, _PERW + 1, _PERW)
        lanes = lax.iota(jnp.int32, _L)

        def transpose_block(src, dst, ncols):
            def rc_step(rc, _):
                col = jnp.full((_L,), 0, jnp.int32) + rc
                for j in range(_D // _L):
                    vals = plsc.load_gather(src, [lanes + j * _L, col])
                    dst[pl.ds(rc * _D + j * _L, _L)] = vals
                return 0
            lax.fori_loop(0, ncols, rc_step, 0, unroll=2)

        def start_in(t, par):
            off = pl.multiple_of((base + t) * 128, 128)
            return pltpu.async_copy(
                tt_hbm.at[:, pl.ds(off, 128)], ib[par], isem[par])

        def wait_in(par):
            pltpu.make_async_copy(
                tt_hbm.at[:, pl.ds(0, 128)], ib[par], isem[par]).wait()

        def start_out(t, par):
            return pltpu.async_copy(
                ob[par], out_hbm.at[pl.ds((base + t) * 128 * _D, 128 * _D)],
                osem[par])

        def wait_out(par):
            pltpu.make_async_copy(
                ob[par], out_hbm.at[pl.ds(0, 128 * _D)], osem[par]).wait()

        @pl.when(nblk > 0)
        def _():
            start_in(0, 0)

        def t2_step(t2, _):
            for par in range(2):
                t = t2 * 2 + par

                @pl.when(t < nblk)
                def _():
                    wait_in(par)

                    @pl.when(t + 1 < nblk)
                    def _():
                        start_in(t + 1, 1 - par)

                    @pl.when(t >= 2)
                    def _():
                        wait_out(par)

                    transpose_block(ib[par], ob[par], 128)
                    start_out(t, par)
            return 0

        lax.fori_loop(0, (_MAXB + 1) // 2, t2_step, 0)

        wait_out(0)
        wait_out(1)

        @pl.when(wid == _NW - 1)
        def _():
            pltpu.sync_copy(tt_hbm.at[:, pl.ds(_NFULL * 128, _TAIL)], tb)
            transpose_block(tb, ob[0], _TAIL)
            pltpu.sync_copy(
                ob[0].at[pl.ds(0, _TAIL * _D)],
                out_hbm.at[pl.ds(_NFULL * 128 * _D, _TAIL * _D)])

    return body(table_t)


def _gather_triplets(idx_i, idx_k, negs, table):
    mesh = plsc.VectorSubcoreMesh(core_axis_name="c", subcore_axis_name="s")

    @functools.partial(
        pl.kernel,
        mesh=mesh,
        out_type=(
            jax.ShapeDtypeStruct((_B, _D), jnp.float32),
            jax.ShapeDtypeStruct((_B, _D), jnp.float32),
            jax.ShapeDtypeStruct((_B, _NEG, _D), jnp.float32),
        ),
        scratch_types=(
            [
                pltpu.VMEM((_CH,), jnp.int32),
                pltpu.VMEM((_CH,), jnp.int32),
                pltpu.VMEM((_NEG, _CH), jnp.int32),
            ]
            + [pltpu.VMEM((_CH, _D), jnp.float32) for _ in range(_NBUF)]
            + [pltpu.SemaphoreType.DMA for _ in range(2 * _NBUF + 1)]
        ),
        compiler_params=pltpu.CompilerParams(
            use_tc_tiling_on_sc=False, needs_layout_passes=False),
    )
    def body(idx_i_hbm, idx_k_hbm, *rest):
        negs_hbm = rest[:_NEG]
        table_hbm, vi_hbm, vk_hbm, vj_hbm = rest[_NEG:_NEG + 4]
        idxi_v, idxk_v, coln_v = rest[_NEG + 4:_NEG + 7]
        bufs = rest[_NEG + 7:_NEG + 7 + _NBUF]
        gsem = rest[_NEG + 7 + _NBUF:_NEG + 7 + 2 * _NBUF]
        wsem = rest[_NEG + 7 + 2 * _NBUF:_NEG + 7 + 3 * _NBUF]
        isem = rest[_NEG + 7 + 3 * _NBUF]

        wid = lax.axis_index("s") * _NC + lax.axis_index("c")
        base = wid * _CH

        ih = [
            pltpu.async_copy(idx_i_hbm.at[pl.ds(base, _CH)], idxi_v, isem),
            pltpu.async_copy(idx_k_hbm.at[pl.ds(base, _CH)], idxk_v, isem),
        ] + [
            pltpu.async_copy(negs_hbm[c].at[pl.ds(base, _CH)],
                             coln_v.at[c], isem)
            for c in range(_NEG)
        ]
        for h in ih:
            h.wait()

        def out2(dst):
            return lambda buf, sem: pltpu.async_copy(
                buf, dst.at[pl.ds(base, _CH)], sem)

        def out3(c):
            return lambda buf, sem: pltpu.async_copy(
                buf, vj_hbm.at[pl.ds(base, _CH), c], sem)

        chunks = [
            (idxi_v, out2(vi_hbm)),
            (idxk_v, out2(vk_hbm)),
        ] + [
            (coln_v.at[c], out3(c)) for c in range(_NEG)
        ]

        gh, wh = {}, {}
        for t in range(_NCHUNK + 1):
            if t < _NCHUNK:
                b = t % _NBUF
                if t >= _NBUF:
                    wh[t - _NBUF].wait()
                gh[t] = pltpu.async_copy(
                    table_hbm.at[chunks[t][0]], bufs[b], gsem[b])
            u = t - 1
            if 0 <= u < _NCHUNK:
                b = u % _NBUF
                gh[u].wait()
                wh[u] = chunks[u][1](bufs[b], wsem[b])
        for u in range(_NCHUNK - _NBUF, _NCHUNK):
            wh[u].wait()

    return body(idx_i, idx_k, *negs, table)


def kernel(items, table):
    items = items.astype(jnp.int32)
    idx_i = items[:, 0]
    idx_k = items[:, 1]
    negs = [items[:, 2 + c] for c in range(_NEG)]
    table_lin = _linearize_table(table.T).reshape(_V, _D)
    return _gather_triplets(idx_i, idx_k, negs, table_lin)


# speedup vs baseline: 1.2719x; 1.2719x over previous
import functools

import jax
import jax.numpy as jnp
from jax import lax
from jax.experimental import pallas as pl
from jax.experimental.pallas import tpu as pltpu
from jax.experimental.pallas import tpu_sc as plsc

_B = 16384
_V = 1000000
_D = 64
_NEG = 8
_NC = 2
_NS = 16
_NW = _NC * _NS
_CH = _B // _NW
_NCHUNK = 2 + _NEG
_NBUF = 3
_L = 16

_NFULL = _V // 128
_TAIL = _V - _NFULL * 128
_PERW = _NFULL // _NW
_EXTRA = _NFULL - _PERW * _NW
_MAXB = _PERW + 1


def _linearize_table(table_t):
    mesh = plsc.VectorSubcoreMesh(core_axis_name="c", subcore_axis_name="s")

    @functools.partial(
        pl.kernel,
        mesh=mesh,
        out_type=jax.ShapeDtypeStruct((_V * _D,), jnp.float32),
        scratch_types=(
            [pltpu.VMEM((_D, 128), jnp.float32) for _ in range(2)]
            + [pltpu.VMEM((128 * _D,), jnp.float32) for _ in range(2)]
            + [pltpu.VMEM((_D, _TAIL), jnp.float32)]
            + [pltpu.SemaphoreType.DMA for _ in range(4)]
        ),
        compiler_params=pltpu.CompilerParams(
            use_tc_tiling_on_sc=True, needs_layout_passes=False),
    )
    def body(tt_hbm, out_hbm, ib0, ib1, ob0, ob1, tb, is0, is1, os0, os1):
        ib, ob, isem, osem = (ib0, ib1), (ob0, ob1), (is0, is1), (os0, os1)
        wid = lax.axis_index("s") * _NC + lax.axis_index("c")
        extra = jnp.minimum(wid, _EXTRA)
        base = wid * _PERW + extra
        nblk = jnp.where(wid < _EXTRA, _PERW + 1, _PERW)
        lanes = lax.iota(jnp.int32, _L)

        zeros = jnp.full((_L,), 0, jnp.int32)
        rowv = [lanes + j * _L for j in range(_D // _L)]

        def transpose_block(src, dst, ncols):
            def rc_step(rc, _):
                cols = [zeros + (rc * 2 + p) for p in range(2)]
                vals = [
                    plsc.load_gather(src, [rowv[j], cols[p]])
                    for p in range(2)
                    for j in range(_D // _L)
                ]
                for p in range(2):
                    for j in range(_D // _L):
                        dst[pl.ds((rc * 2 + p) * _D + j * _L, _L)] = (
                            vals[p * (_D // _L) + j])
                return 0
            lax.fori_loop(0, ncols // 2, rc_step, 0, unroll=8)

        def start_in(t, par):
            off = pl.multiple_of((base + t) * 128, 128)
            return pltpu.async_copy(
                tt_hbm.at[:, pl.ds(off, 128)], ib[par], isem[par])

        def wait_in(par):
            pltpu.make_async_copy(
                tt_hbm.at[:, pl.ds(0, 128)], ib[par], isem[par]).wait()

        def start_out(t, par):
            return pltpu.async_copy(
                ob[par], out_hbm.at[pl.ds((base + t) * 128 * _D, 128 * _D)],
                osem[par])

        def wait_out(par):
            pltpu.make_async_copy(
                ob[par], out_hbm.at[pl.ds(0, 128 * _D)], osem[par]).wait()

        @pl.when(nblk > 0)
        def _():
            start_in(0, 0)

        def t2_step(t2, _):
            for par in range(2):
                t = t2 * 2 + par

                @pl.when(t < nblk)
                def _():
                    wait_in(par)

                    @pl.when(t + 1 < nblk)
                    def _():
                        start_in(t + 1, 1 - par)

                    @pl.when(t >= 2)
                    def _():
                        wait_out(par)

                    transpose_block(ib[par], ob[par], 128)
                    start_out(t, par)
            return 0

        lax.fori_loop(0, (_MAXB + 1) // 2, t2_step, 0)

        wait_out(0)
        wait_out(1)

        @pl.when(wid == _NW - 1)
        def _():
            pltpu.sync_copy(tt_hbm.at[:, pl.ds(_NFULL * 128, _TAIL)], tb)
            transpose_block(tb, ob[0], _TAIL)
            pltpu.sync_copy(
                ob[0].at[pl.ds(0, _TAIL * _D)],
                out_hbm.at[pl.ds(_NFULL * 128 * _D, _TAIL * _D)])

    return body(table_t)


def _gather_triplets(idx_i, idx_k, negs, table):
    mesh = plsc.VectorSubcoreMesh(core_axis_name="c", subcore_axis_name="s")

    @functools.partial(
        pl.kernel,
        mesh=mesh,
        out_type=(
            jax.ShapeDtypeStruct((_B, _D), jnp.float32),
            jax.ShapeDtypeStruct((_B, _D), jnp.float32),
            jax.ShapeDtypeStruct((_B, _NEG, _D), jnp.float32),
        ),
        scratch_types=(
            [
                pltpu.VMEM((_CH,), jnp.int32),
                pltpu.VMEM((_CH,), jnp.int32),
                pltpu.VMEM((_NEG, _CH), jnp.int32),
            ]
            + [pltpu.VMEM((_CH, _D), jnp.float32) for _ in range(_NBUF)]
            + [pltpu.SemaphoreType.DMA for _ in range(2 * _NBUF + 1)]
        ),
        compiler_params=pltpu.CompilerParams(
            use_tc_tiling_on_sc=False, needs_layout_passes=False),
    )
    def body(idx_i_hbm, idx_k_hbm, *rest):
        negs_hbm = rest[:_NEG]
        table_hbm, vi_hbm, vk_hbm, vj_hbm = rest[_NEG:_NEG + 4]
        idxi_v, idxk_v, coln_v = rest[_NEG + 4:_NEG + 7]
        bufs = rest[_NEG + 7:_NEG + 7 + _NBUF]
        gsem = rest[_NEG + 7 + _NBUF:_NEG + 7 + 2 * _NBUF]
        wsem = rest[_NEG + 7 + 2 * _NBUF:_NEG + 7 + 3 * _NBUF]
        isem = rest[_NEG + 7 + 3 * _NBUF]

        wid = lax.axis_index("s") * _NC + lax.axis_index("c")
        base = wid * _CH

        ih = [
            pltpu.async_copy(idx_i_hbm.at[pl.ds(base, _CH)], idxi_v, isem),
            pltpu.async_copy(idx_k_hbm.at[pl.ds(base, _CH)], idxk_v, isem),
        ] + [
            pltpu.async_copy(negs_hbm[c].at[pl.ds(base, _CH)],
                             coln_v.at[c], isem)
            for c in range(_NEG)
        ]
        for h in ih:
            h.wait()

        def out2(dst):
            return lambda buf, sem: pltpu.async_copy(
                buf, dst.at[pl.ds(base, _CH)], sem)

        def out3(c):
            return lambda buf, sem: pltpu.async_copy(
                buf, vj_hbm.at[pl.ds(base, _CH), c], sem)

        chunks = [
            (idxi_v, out2(vi_hbm)),
            (idxk_v, out2(vk_hbm)),
        ] + [
            (coln_v.at[c], out3(c)) for c in range(_NEG)
        ]

        gh, wh = {}, {}
        for t in range(_NCHUNK + 1):
            if t < _NCHUNK:
                b = t % _NBUF
                if t >= _NBUF:
                    wh[t - _NBUF].wait()
                gh[t] = pltpu.async_copy(
                    table_hbm.at[chunks[t][0]], bufs[b], gsem[b])
            u = t - 1
            if 0 <= u < _NCHUNK:
                b = u % _NBUF
                gh[u].wait()
                wh[u] = chunks[u][1](bufs[b], wsem[b])
        for u in range(_NCHUNK - _NBUF, _NCHUNK):
            wh[u].wait()

    return body(idx_i, idx_k, *negs, table)


def kernel(items, table):
    items = items.astype(jnp.int32)
    idx_i = items[:, 0]
    idx_k = items[:, 1]
    negs = [items[:, 2 + c] for c in range(_NEG)]
    table_lin = _linearize_table(table.T).reshape(_V, _D)
    return _gather_triplets(idx_i, idx_k, negs, table_lin)


# speedup vs baseline: 2.2324x; 1.7552x over previous
import functools

import jax
import jax.numpy as jnp
from jax import lax
from jax.experimental import pallas as pl
from jax.experimental.pallas import tpu as pltpu
from jax.experimental.pallas import tpu_sc as plsc

_B = 16384
_D = 64
_NEG = 8
_NC = 2
_NS = 16
_NW = _NC * _NS
_CH = _B // _NW
_NCHUNK = 2 + _NEG
_NBUF = 3


def _gather_triplets(idx_i, idx_k, negs, table):
    mesh = plsc.VectorSubcoreMesh(core_axis_name="c", subcore_axis_name="s")

    @functools.partial(
        pl.kernel,
        mesh=mesh,
        out_type=(
            jax.ShapeDtypeStruct((_B, _D), jnp.float32),
            jax.ShapeDtypeStruct((_B, _D), jnp.float32),
            jax.ShapeDtypeStruct((_B, _NEG, _D), jnp.float32),
        ),
        scratch_types=(
            [
                pltpu.VMEM((_CH,), jnp.int32),
                pltpu.VMEM((_CH,), jnp.int32),
                pltpu.VMEM((_NEG, _CH), jnp.int32),
            ]
            + [pltpu.VMEM((_CH, _D), jnp.float32) for _ in range(_NBUF)]
            + [pltpu.SemaphoreType.DMA for _ in range(2 * _NBUF + 1)]
        ),
        compiler_params=pltpu.CompilerParams(
            use_tc_tiling_on_sc=False, needs_layout_passes=False),
    )
    def body(idx_i_hbm, idx_k_hbm, *rest):
        negs_hbm = rest[:_NEG]
        table_hbm, vi_hbm, vk_hbm, vj_hbm = rest[_NEG:_NEG + 4]
        idxi_v, idxk_v, coln_v = rest[_NEG + 4:_NEG + 7]
        bufs = rest[_NEG + 7:_NEG + 7 + _NBUF]
        gsem = rest[_NEG + 7 + _NBUF:_NEG + 7 + 2 * _NBUF]
        wsem = rest[_NEG + 7 + 2 * _NBUF:_NEG + 7 + 3 * _NBUF]
        isem = rest[_NEG + 7 + 3 * _NBUF]

        wid = lax.axis_index("s") * _NC + lax.axis_index("c")
        base = wid * _CH

        ih = [
            pltpu.async_copy(idx_i_hbm.at[pl.ds(base, _CH)], idxi_v, isem),
            pltpu.async_copy(idx_k_hbm.at[pl.ds(base, _CH)], idxk_v, isem),
        ] + [
            pltpu.async_copy(negs_hbm[c].at[pl.ds(base, _CH)],
                             coln_v.at[c], isem)
            for c in range(_NEG)
        ]
        for h in ih:
            h.wait()

        def out2(dst):
            return lambda buf, sem: pltpu.async_copy(
                buf, dst.at[pl.ds(base, _CH)], sem)

        def out3(c):
            return lambda buf, sem: pltpu.async_copy(
                buf, vj_hbm.at[pl.ds(base, _CH), c], sem)

        chunks = [
            (idxi_v, out2(vi_hbm)),
            (idxk_v, out2(vk_hbm)),
        ] + [
            (coln_v.at[c], out3(c)) for c in range(_NEG)
        ]

        gh, wh = {}, {}
        for t in range(_NCHUNK + 1):
            if t < _NCHUNK:
                b = t % _NBUF
                if t >= _NBUF:
                    wh[t - _NBUF].wait()
                gh[t] = pltpu.async_copy(
                    table_hbm.at[chunks[t][0]], bufs[b], gsem[b])
            u = t - 1
            if 0 <= u < _NCHUNK:
                b = u % _NBUF
                gh[u].wait()
                wh[u] = chunks[u][1](bufs[b], wsem[b])
        for u in range(_NCHUNK - _NBUF, _NCHUNK):
            wh[u].wait()

    return body(idx_i, idx_k, *negs, table)


def kernel(items, table):
    items = items.astype(jnp.int32)
    idx_i = items[:, 0]
    idx_k = items[:, 1]
    negs = [items[:, 2 + c] for c in range(_NEG)]
    return _gather_triplets(idx_i, idx_k, negs, table)
